# baseline (speedup 1.0000x reference)
import jax
import jax.numpy as jnp
from jax.experimental import pallas as pl
from jax.experimental.pallas import tpu as pltpu

DIM = 192
NUM_HEADS = 4
HEAD_DIM = 48
N_WIN = 28
RS = 8
NR = N_WIN * N_WIN
RSS = RS * RS
TOPK = 4
SCALE = DIM ** (-0.5)
ROWS = NR * RSS
H = W = 224

RA = 3584


def _qkv_body(x_ref, w_ref, b_ref, q4_ref, k4_ref, v4_ref, vsp_ref,
              qr_ref, kr_ref):
    y = jax.lax.dot_general(x_ref[:], w_ref[:], (((1,), (0,)), ((), ())),
                            preferred_element_type=jnp.float32)
    y = y.reshape(RA, 3 * DIM) + b_ref[0]
    y16 = y.astype(jnp.bfloat16)
    vsp_ref[:] = y16[:, 2 * DIM:]
    for h in range(NUM_HEADS):
        q4_ref[h] = y16[:, h * HEAD_DIM:(h + 1) * HEAD_DIM]
        k4_ref[h] = y16[:, DIM + h * HEAD_DIM:DIM + (h + 1) * HEAD_DIM]
        v4_ref[h] = y16[:, 2 * DIM + h * HEAD_DIM:2 * DIM + (h + 1) * HEAD_DIM]
    pooled = jnp.mean(y[:, :2 * DIM].reshape(2, RS, N_WIN, RS, 2 * DIM),
                      axis=(1, 3)).reshape(2 * N_WIN, 2 * DIM)
    qr_ref[:] = pooled[:, :DIM]
    kr_ref[:] = pooled[:, DIM:]


def _qkv_proj(x2d, wT, b):
    return pl.pallas_call(
        _qkv_body,
        grid=(ROWS // RA,),
        in_specs=[
            pl.BlockSpec((RA // W, DIM, W), lambda i: (i, 0, 0)),
            pl.BlockSpec((DIM, 3 * DIM), lambda i: (0, 0)),
            pl.BlockSpec((1, 3 * DIM), lambda i: (0, 0)),
        ],
        out_specs=[
            pl.BlockSpec((NUM_HEADS, RA, HEAD_DIM), lambda i: (0, i, 0)),
            pl.BlockSpec((NUM_HEADS, RA, HEAD_DIM), lambda i: (0, i, 0)),
            pl.BlockSpec((NUM_HEADS, RA, HEAD_DIM), lambda i: (0, i, 0)),
            pl.BlockSpec((RA, DIM), lambda i: (i, 0)),
            pl.BlockSpec((2 * N_WIN, DIM), lambda i: (i, 0)),
            pl.BlockSpec((2 * N_WIN, DIM), lambda i: (i, 0)),
        ],
        out_shape=[
            jax.ShapeDtypeStruct((NUM_HEADS, ROWS, HEAD_DIM), jnp.bfloat16),
            jax.ShapeDtypeStruct((NUM_HEADS, ROWS, HEAD_DIM), jnp.bfloat16),
            jax.ShapeDtypeStruct((NUM_HEADS, ROWS, HEAD_DIM), jnp.bfloat16),
            jax.ShapeDtypeStruct((ROWS, DIM), jnp.bfloat16),
            jax.ShapeDtypeStruct((NR, DIM), jnp.float32),
            jax.ShapeDtypeStruct((NR, DIM), jnp.float32),
        ],
    )(x2d, wT, b)


def _route_body(qr_ref, kr_ref, ri_ref, rc_ref):
    a = jax.lax.dot_general(qr_ref[:], kr_ref[:], (((1,), (1,)), ((), ())),
                            preferred_element_type=jnp.float32)
    iota = jax.lax.broadcasted_iota(jnp.int32, a.shape, 1)
    for t in range(TOPK):
        m = jnp.max(a, axis=1, keepdims=True)
        ii = jnp.min(jnp.where(a == m, iota, NR), axis=1)
        ji = ii // N_WIN
        ri_ref[t] = ji * RS
        rc_ref[t] = (ii - ji * N_WIN) * RS
        a = jnp.where(iota == ii[:, None], -1e30, a)


def _route(qr, kr):
    return pl.pallas_call(
        _route_body,
        grid=(1,),
        in_specs=[
            pl.BlockSpec((NR, DIM), lambda i: (0, 0)),
            pl.BlockSpec((NR, DIM), lambda i: (0, 0)),
        ],
        out_specs=[
            pl.BlockSpec((TOPK, NR), lambda i: (0, 0)),
            pl.BlockSpec((TOPK, NR), lambda i: (0, 0)),
        ],
        out_shape=[
            jax.ShapeDtypeStruct((TOPK, NR), jnp.int32),
            jax.ShapeDtypeStruct((TOPK, NR), jnp.int32),
        ],
    )(qr, kr)


def _attn_body(ri_ref, rc_ref, q_ref, k_ref, v_ref, o_ref):
    i = pl.program_id(1)

    def region_tile(ref, ro, co):
        t = ref[0, pl.ds(ro, RS), pl.ds(co, RS), :]
        return t.reshape(RSS, HEAD_DIM)

    kgs, vgs = [], []
    for j in range(N_WIN):
        r = i * N_WIN + j
        offs = [(pl.multiple_of(ri_ref[t, r], RS),
                 pl.multiple_of(rc_ref[t, r], RS)) for t in range(TOPK)]
        kgs.append(jnp.concatenate(
            [region_tile(k_ref, ro, co) for ro, co in offs], axis=0))
        vgs.append(jnp.concatenate(
            [region_tile(v_ref, ro, co) for ro, co in offs], axis=0))
    KG = jnp.stack(kgs, axis=0)
    VG = jnp.stack(vgs, axis=0)
    Q = (q_ref[0].reshape(RS, N_WIN, RS, HEAD_DIM)
         .transpose(1, 0, 2, 3).reshape(N_WIN, RSS, HEAD_DIM))
    S = jax.lax.dot_general(Q, KG, (((2,), (2,)), ((0,), (0,))),
                            preferred_element_type=jnp.float32)
    m = jnp.max(S, axis=2, keepdims=True)
    e = jnp.exp(S - m)
    ssum = jnp.sum(e, axis=2, keepdims=True)
    O = jax.lax.dot_general(e.astype(jnp.bfloat16), VG,
                            (((2,), (1,)), ((0,), (0,))),
                            preferred_element_type=jnp.float32) / ssum
    o_ref[0] = (O.astype(jnp.bfloat16)
                .reshape(N_WIN, RS, RS, HEAD_DIM)
                .transpose(1, 0, 2, 3).reshape(RS, W, HEAD_DIM))


def _attention(ri, rc, q4, k4, v4):
    grid_spec = pltpu.PrefetchScalarGridSpec(
        num_scalar_prefetch=2,
        grid=(NUM_HEADS, N_WIN),
        in_specs=[
            pl.BlockSpec((1, RS, W, HEAD_DIM), lambda h, i, *_: (h, i, 0, 0)),
            pl.BlockSpec((1, H, W, HEAD_DIM), lambda h, i, *_: (h, 0, 0, 0)),
            pl.BlockSpec((1, H, W, HEAD_DIM), lambda h, i, *_: (h, 0, 0, 0)),
        ],
        out_specs=pl.BlockSpec((1, RS, W, HEAD_DIM), lambda h, i, *_: (h, i, 0, 0)),
    )
    return pl.pallas_call(
        _attn_body,
        grid_spec=grid_spec,
        out_shape=jax.ShapeDtypeStruct((NUM_HEADS, H, W, HEAD_DIM), jnp.bfloat16),
    )(ri, rc, q4, k4, v4)


RCH = 8


def _out_body(a_ref, vp_ref, wh_ref, wl_ref, w9_ref, lb_ref, b_ref, o_ref,
              buf, sem):
    i = pl.program_id(0)

    def dma(slot, blk):
        return pltpu.make_async_copy(vp_ref.at[pl.ds(blk * RCH, RCH + 2)],
                                     buf.at[slot], sem.at[slot])

    @pl.when(i == 0)
    def _():
        dma(0, 0).start()

    @pl.when(i + 1 < H // RCH)
    def _():
        dma((i + 1) % 2, i + 1).start()

    dma(i % 2, i).wait()
    rows10 = buf[i % 2].astype(jnp.float32)
    lep = jnp.zeros((RCH, W, DIM), jnp.float32) + lb_ref[0]
    for dy in range(3):
        for dx in range(3):
            lep = lep + rows10[dy:dy + RCH, dx:dx + W, :] * w9_ref[dy * 3 + dx]
    l16 = lep.astype(jnp.bfloat16)
    for h in range(RCH):
        acc = jax.lax.dot_general(wl_ref[:], l16[h], (((0,), (1,)), ((), ())),
                                  preferred_element_type=jnp.float32)
        for hd in range(NUM_HEADS):
            acc = acc + jax.lax.dot_general(
                wh_ref[hd], a_ref[hd, h], (((0,), (1,)), ((), ())),
                preferred_element_type=jnp.float32)
        o_ref[h] = acc + b_ref[:]


def _out_proj(attn4, vp, wh, wl, w9, lb, b):
    return pl.pallas_call(
        _out_body,
        grid=(H // RCH,),
        in_specs=[
            pl.BlockSpec((NUM_HEADS, RCH, W, HEAD_DIM), lambda i: (0, i, 0, 0)),
            pl.BlockSpec(memory_space=pl.ANY),
            pl.BlockSpec((NUM_HEADS, HEAD_DIM, DIM), lambda i: (0, 0, 0)),
            pl.BlockSpec((DIM, DIM), lambda i: (0, 0)),
            pl.BlockSpec((9, DIM), lambda i: (0, 0)),
            pl.BlockSpec((1, DIM), lambda i: (0, 0)),
            pl.BlockSpec((DIM, 1), lambda i: (0, 0)),
        ],
        out_specs=pl.BlockSpec((RCH, DIM, W), lambda i: (i, 0, 0)),
        out_shape=jax.ShapeDtypeStruct((H, DIM, W), jnp.float32),
        scratch_shapes=[
            pltpu.VMEM((2, RCH + 2, W + 2, DIM), jnp.bfloat16),
            pltpu.SemaphoreType.DMA((2,)),
        ],
    )(attn4, vp, wh, wl, w9, lb, b)


def kernel(x, qkv_w, qkv_b, lepe_w, lepe_b, out_w, out_b):
    x_t = jnp.transpose(x, (0, 1, 3, 2)).reshape(H, DIM, W)

    w_s = jnp.concatenate([qkv_w[:DIM] * SCALE, qkv_w[DIM:]], axis=0)
    b_s = jnp.concatenate([qkv_b[:DIM] * SCALE, qkv_b[DIM:]])
    q4, k4, v4, v_sp, qr, kr = _qkv_proj(x_t, w_s.T, b_s.reshape(1, -1))
    ri, rc = _route(qr, kr)

    q4 = q4.reshape(NUM_HEADS, H, W, HEAD_DIM)
    k4 = k4.reshape(NUM_HEADS, H, W, HEAD_DIM)
    v4 = v4.reshape(NUM_HEADS, H, W, HEAD_DIM)
    attn4 = _attention(ri, rc, q4, k4, v4)

    vp = jnp.pad(v_sp.reshape(H, W, DIM), ((1, 15), (1, 1), (0, 0)))

    wT16 = out_w.T.astype(jnp.bfloat16)
    wh = wT16.reshape(NUM_HEADS, HEAD_DIM, DIM)
    out = _out_proj(attn4, vp, wh, wT16, lepe_w.reshape(DIM, 9).T,
                    lepe_b.reshape(1, -1), out_b.reshape(-1, 1))
    return jnp.transpose(out.reshape(1, H, DIM, W), (0, 1, 3, 2))


# speedup vs baseline: 1.0151x; 1.0151x over previous
import jax
import jax.numpy as jnp
from jax.experimental import pallas as pl
from jax.experimental.pallas import tpu as pltpu

DIM = 192
NUM_HEADS = 4
HEAD_DIM = 48
N_WIN = 28
RS = 8
NR = N_WIN * N_WIN
RSS = RS * RS
TOPK = 4
SCALE = DIM ** (-0.5)
ROWS = NR * RSS
H = W = 224

RA = 3584


def _qkv_body(x_ref, w_ref, b_ref, q4_ref, k4_ref, v4_ref, vsp_ref,
              qr_ref, kr_ref):
    y = jax.lax.dot_general(x_ref[:], w_ref[:], (((1,), (0,)), ((), ())),
                            preferred_element_type=jnp.float32)
    y = y.reshape(RA, 3 * DIM) + b_ref[0]
    y16 = y.astype(jnp.bfloat16)
    vsp_ref[:] = y16[:, 2 * DIM:]
    yq = (y[:, :DIM] * SCALE).astype(jnp.bfloat16)
    for h in range(NUM_HEADS):
        q4_ref[h] = yq[:, h * HEAD_DIM:(h + 1) * HEAD_DIM]
        k4_ref[h] = y16[:, DIM + h * HEAD_DIM:DIM + (h + 1) * HEAD_DIM]
        v4_ref[h] = y16[:, 2 * DIM + h * HEAD_DIM:2 * DIM + (h + 1) * HEAD_DIM]
    pooled = jnp.mean(y[:, :2 * DIM].reshape(2, RS, N_WIN, RS, 2 * DIM),
                      axis=(1, 3)).reshape(2 * N_WIN, 2 * DIM)
    qr_ref[:] = pooled[:, :DIM]
    kr_ref[:] = pooled[:, DIM:]


def _qkv_proj(x2d, wT, b):
    return pl.pallas_call(
        _qkv_body,
        grid=(ROWS // RA,),
        in_specs=[
            pl.BlockSpec((RA // W, DIM, W), lambda i: (i, 0, 0)),
            pl.BlockSpec((DIM, 3 * DIM), lambda i: (0, 0)),
            pl.BlockSpec((1, 3 * DIM), lambda i: (0, 0)),
        ],
        out_specs=[
            pl.BlockSpec((NUM_HEADS, RA, HEAD_DIM), lambda i: (0, i, 0)),
            pl.BlockSpec((NUM_HEADS, RA, HEAD_DIM), lambda i: (0, i, 0)),
            pl.BlockSpec((NUM_HEADS, RA, HEAD_DIM), lambda i: (0, i, 0)),
            pl.BlockSpec((RA, DIM), lambda i: (i, 0)),
            pl.BlockSpec((2 * N_WIN, DIM), lambda i: (i, 0)),
            pl.BlockSpec((2 * N_WIN, DIM), lambda i: (i, 0)),
        ],
        out_shape=[
            jax.ShapeDtypeStruct((NUM_HEADS, ROWS, HEAD_DIM), jnp.bfloat16),
            jax.ShapeDtypeStruct((NUM_HEADS, ROWS, HEAD_DIM), jnp.bfloat16),
            jax.ShapeDtypeStruct((NUM_HEADS, ROWS, HEAD_DIM), jnp.bfloat16),
            jax.ShapeDtypeStruct((ROWS, DIM), jnp.bfloat16),
            jax.ShapeDtypeStruct((NR, DIM), jnp.float32),
            jax.ShapeDtypeStruct((NR, DIM), jnp.float32),
        ],
    )(x2d, wT, b)


def _route_body(qr_ref, kr_ref, ri_ref, rc_ref):
    a = jax.lax.dot_general(qr_ref[:], kr_ref[:], (((1,), (1,)), ((), ())),
                            preferred_element_type=jnp.float32)
    iota = jax.lax.broadcasted_iota(jnp.int32, a.shape, 1)
    for t in range(TOPK):
        m = jnp.max(a, axis=1, keepdims=True)
        ii = jnp.min(jnp.where(a == m, iota, NR), axis=1)
        ji = ii // N_WIN
        ri_ref[t] = ji * RS
        rc_ref[t] = (ii - ji * N_WIN) * RS
        a = jnp.where(iota == ii[:, None], -1e30, a)


def _route(qr, kr):
    return pl.pallas_call(
        _route_body,
        grid=(1,),
        in_specs=[
            pl.BlockSpec((NR, DIM), lambda i: (0, 0)),
            pl.BlockSpec((NR, DIM), lambda i: (0, 0)),
        ],
        out_specs=[
            pl.BlockSpec((TOPK, NR), lambda i: (0, 0)),
            pl.BlockSpec((TOPK, NR), lambda i: (0, 0)),
        ],
        out_shape=[
            jax.ShapeDtypeStruct((TOPK, NR), jnp.int32),
            jax.ShapeDtypeStruct((TOPK, NR), jnp.int32),
        ],
    )(qr, kr)


def _attn_body(ri_ref, rc_ref, q_ref, k_ref, v_ref, o_ref):
    i = pl.program_id(1)

    def region_tile(ref, ro, co):
        t = ref[0, pl.ds(ro, RS), pl.ds(co, RS), :]
        return t.reshape(RSS, HEAD_DIM)

    kgs, vgs = [], []
    for j in range(N_WIN):
        r = i * N_WIN + j
        offs = [(pl.multiple_of(ri_ref[t, r], RS),
                 pl.multiple_of(rc_ref[t, r], RS)) for t in range(TOPK)]
        kgs.append(jnp.concatenate(
            [region_tile(k_ref, ro, co) for ro, co in offs], axis=0))
        vgs.append(jnp.concatenate(
            [region_tile(v_ref, ro, co) for ro, co in offs], axis=0))
    KG = jnp.stack(kgs, axis=0)
    VG = jnp.stack(vgs, axis=0)
    Q = (q_ref[0].reshape(RS, N_WIN, RS, HEAD_DIM)
         .transpose(1, 0, 2, 3).reshape(N_WIN, RSS, HEAD_DIM))
    S = jax.lax.dot_general(Q, KG, (((2,), (2,)), ((0,), (0,))),
                            preferred_element_type=jnp.float32)
    m = jnp.max(S, axis=2, keepdims=True)
    e = jnp.exp(S - m)
    ssum = jnp.sum(e, axis=2, keepdims=True)
    O = jax.lax.dot_general(e.astype(jnp.bfloat16), VG,
                            (((2,), (1,)), ((0,), (0,))),
                            preferred_element_type=jnp.float32) / ssum
    o_ref[0] = (O.astype(jnp.bfloat16)
                .reshape(N_WIN, RS, RS, HEAD_DIM)
                .transpose(1, 0, 2, 3).reshape(RS, W, HEAD_DIM))


def _attention(ri, rc, q4, k4, v4):
    grid_spec = pltpu.PrefetchScalarGridSpec(
        num_scalar_prefetch=2,
        grid=(NUM_HEADS, N_WIN),
        in_specs=[
            pl.BlockSpec((1, RS, W, HEAD_DIM), lambda h, i, *_: (h, i, 0, 0)),
            pl.BlockSpec((1, H, W, HEAD_DIM), lambda h, i, *_: (h, 0, 0, 0)),
            pl.BlockSpec((1, H, W, HEAD_DIM), lambda h, i, *_: (h, 0, 0, 0)),
        ],
        out_specs=pl.BlockSpec((1, RS, W, HEAD_DIM), lambda h, i, *_: (h, i, 0, 0)),
    )
    return pl.pallas_call(
        _attn_body,
        grid_spec=grid_spec,
        out_shape=jax.ShapeDtypeStruct((NUM_HEADS, H, W, HEAD_DIM), jnp.bfloat16),
    )(ri, rc, q4, k4, v4)


RCH = 8


def _out_body(a_ref, vp_ref, wh_ref, wl_ref, w9_ref, lb_ref, b_ref, o_ref,
              buf, sem):
    i = pl.program_id(0)

    def dma(slot, blk):
        return pltpu.make_async_copy(vp_ref.at[pl.ds(blk * RCH, RCH + 2)],
                                     buf.at[slot], sem.at[slot])

    @pl.when(i == 0)
    def _():
        dma(0, 0).start()

    @pl.when(i + 1 < H // RCH)
    def _():
        dma((i + 1) % 2, i + 1).start()

    dma(i % 2, i).wait()
    rows10 = buf[i % 2].astype(jnp.float32)
    lep = jnp.zeros((RCH, W, DIM), jnp.float32) + lb_ref[0]
    for dy in range(3):
        for dx in range(3):
            lep = lep + rows10[dy:dy + RCH, dx:dx + W, :] * w9_ref[dy * 3 + dx]
    l16 = lep.astype(jnp.bfloat16)
    for h in range(RCH):
        acc = jax.lax.dot_general(wl_ref[:], l16[h], (((0,), (1,)), ((), ())),
                                  preferred_element_type=jnp.float32)
        for hd in range(NUM_HEADS):
            acc = acc + jax.lax.dot_general(
                wh_ref[hd], a_ref[hd, h], (((0,), (1,)), ((), ())),
                preferred_element_type=jnp.float32)
        o_ref[h] = acc + b_ref[:]


def _out_proj(attn4, vp, wh, wl, w9, lb, b):
    return pl.pallas_call(
        _out_body,
        grid=(H // RCH,),
        in_specs=[
            pl.BlockSpec((NUM_HEADS, RCH, W, HEAD_DIM), lambda i: (0, i, 0, 0)),
            pl.BlockSpec(memory_space=pl.ANY),
            pl.BlockSpec((NUM_HEADS, HEAD_DIM, DIM), lambda i: (0, 0, 0)),
            pl.BlockSpec((DIM, DIM), lambda i: (0, 0)),
            pl.BlockSpec((9, DIM), lambda i: (0, 0)),
            pl.BlockSpec((1, DIM), lambda i: (0, 0)),
            pl.BlockSpec((DIM, 1), lambda i: (0, 0)),
        ],
        out_specs=pl.BlockSpec((RCH, DIM, W), lambda i: (i, 0, 0)),
        out_shape=jax.ShapeDtypeStruct((H, DIM, W), jnp.float32),
        scratch_shapes=[
            pltpu.VMEM((2, RCH + 2, W + 2, DIM), jnp.bfloat16),
            pltpu.SemaphoreType.DMA((2,)),
        ],
    )(attn4, vp, wh, wl, w9, lb, b)


def kernel(x, qkv_w, qkv_b, lepe_w, lepe_b, out_w, out_b):
    x_t = jnp.transpose(x, (0, 1, 3, 2)).reshape(H, DIM, W)

    q4, k4, v4, v_sp, qr, kr = _qkv_proj(x_t, qkv_w.T, qkv_b.reshape(1, -1))
    ri, rc = _route(qr, kr)

    q4 = q4.reshape(NUM_HEADS, H, W, HEAD_DIM)
    k4 = k4.reshape(NUM_HEADS, H, W, HEAD_DIM)
    v4 = v4.reshape(NUM_HEADS, H, W, HEAD_DIM)
    attn4 = _attention(ri, rc, q4, k4, v4)

    vp = jnp.pad(v_sp.reshape(H, W, DIM), ((1, 15), (1, 1), (0, 0)))

    wT16 = out_w.T.astype(jnp.bfloat16)
    wh = wT16.reshape(NUM_HEADS, HEAD_DIM, DIM)
    out = _out_proj(attn4, vp, wh, wT16, lepe_w.reshape(DIM, 9).T,
                    lepe_b.reshape(1, -1), out_b.reshape(-1, 1))
    return jnp.transpose(out.reshape(1, H, DIM, W), (0, 1, 3, 2))


# speedup vs baseline: 1.0450x; 1.0295x over previous
import jax
import jax.numpy as jnp
from jax.experimental import pallas as pl
from jax.experimental.pallas import tpu as pltpu

DIM = 192
NUM_HEADS = 4
HEAD_DIM = 48
N_WIN = 28
RS = 8
NR = N_WIN * N_WIN
RSS = RS * RS
TOPK = 4
SCALE = DIM ** (-0.5)
ROWS = NR * RSS
H = W = 224

RA = 3584


def _qkv_body(x_ref, w_ref, b_ref, q4_ref, k4_ref, v4_ref, vsp_ref,
              qr_ref, kr_ref):
    y = jax.lax.dot_general(x_ref[:], w_ref[:], (((1,), (0,)), ((), ())),
                            preferred_element_type=jnp.float32)
    y = y.reshape(RA, 3 * DIM) + b_ref[0]
    y16 = y.astype(jnp.bfloat16)
    vsp_ref[:] = y16[:, 2 * DIM:]
    yq = (y[:, :DIM] * SCALE).astype(jnp.bfloat16)
    for h in range(NUM_HEADS):
        q4_ref[h] = yq[:, h * HEAD_DIM:(h + 1) * HEAD_DIM]
        k4_ref[h] = y16[:, DIM + h * HEAD_DIM:DIM + (h + 1) * HEAD_DIM]
        v4_ref[h] = y16[:, 2 * DIM + h * HEAD_DIM:2 * DIM + (h + 1) * HEAD_DIM]
    pooled = jnp.mean(y[:, :2 * DIM].reshape(2, RS, N_WIN, RS, 2 * DIM),
                      axis=(1, 3)).reshape(2 * N_WIN, 2 * DIM)
    qr_ref[:] = pooled[:, :DIM]
    kr_ref[:] = pooled[:, DIM:]


def _qkv_proj(x2d, wT, b):
    return pl.pallas_call(
        _qkv_body,
        grid=(ROWS // RA,),
        in_specs=[
            pl.BlockSpec((RA // W, DIM, W), lambda i: (i, 0, 0)),
            pl.BlockSpec((DIM, 3 * DIM), lambda i: (0, 0)),
            pl.BlockSpec((1, 3 * DIM), lambda i: (0, 0)),
        ],
        out_specs=[
            pl.BlockSpec((NUM_HEADS, RA, HEAD_DIM), lambda i: (0, i, 0)),
            pl.BlockSpec((NUM_HEADS, RA, HEAD_DIM), lambda i: (0, i, 0)),
            pl.BlockSpec((NUM_HEADS, RA, HEAD_DIM), lambda i: (0, i, 0)),
            pl.BlockSpec((RA, DIM), lambda i: (i, 0)),
            pl.BlockSpec((2 * N_WIN, DIM), lambda i: (i, 0)),
            pl.BlockSpec((2 * N_WIN, DIM), lambda i: (i, 0)),
        ],
        out_shape=[
            jax.ShapeDtypeStruct((NUM_HEADS, ROWS, HEAD_DIM), jnp.bfloat16),
            jax.ShapeDtypeStruct((NUM_HEADS, ROWS, HEAD_DIM), jnp.bfloat16),
            jax.ShapeDtypeStruct((NUM_HEADS, ROWS, HEAD_DIM), jnp.bfloat16),
            jax.ShapeDtypeStruct((ROWS, DIM), jnp.bfloat16),
            jax.ShapeDtypeStruct((NR, DIM), jnp.float32),
            jax.ShapeDtypeStruct((NR, DIM), jnp.float32),
        ],
    )(x2d, wT, b)


def _route_body(qr_ref, kr_ref, ri_ref, rc_ref):
    a = jax.lax.dot_general(qr_ref[:], kr_ref[:], (((1,), (1,)), ((), ())),
                            preferred_element_type=jnp.float32)
    iota = jax.lax.broadcasted_iota(jnp.int32, a.shape, 1)
    for t in range(TOPK):
        m = jnp.max(a, axis=1, keepdims=True)
        ii = jnp.min(jnp.where(a == m, iota, NR), axis=1)
        ji = ii // N_WIN
        ri_ref[t] = ji * RS
        rc_ref[t] = (ii - ji * N_WIN) * RS
        a = jnp.where(iota == ii[:, None], -1e30, a)


def _route(qr, kr):
    return pl.pallas_call(
        _route_body,
        grid=(1,),
        in_specs=[
            pl.BlockSpec((NR, DIM), lambda i: (0, 0)),
            pl.BlockSpec((NR, DIM), lambda i: (0, 0)),
        ],
        out_specs=[
            pl.BlockSpec((TOPK, NR), lambda i: (0, 0)),
            pl.BlockSpec((TOPK, NR), lambda i: (0, 0)),
        ],
        out_shape=[
            jax.ShapeDtypeStruct((TOPK, NR), jnp.int32),
            jax.ShapeDtypeStruct((TOPK, NR), jnp.int32),
        ],
    )(qr, kr)


BANDS = 2


def _attn_body(ri_ref, rc_ref, q_ref, k_ref, v_ref, o_ref):
    i = pl.program_id(1)

    def region_tile(ref, ro, co):
        t = ref[0, pl.ds(ro, RS), pl.ds(co, RS), :]
        return t.reshape(RSS, HEAD_DIM)

    nreg = N_WIN * BANDS
    kgs, vgs = [], []
    for j in range(nreg):
        r = i * nreg + j
        offs = [(pl.multiple_of(ri_ref[t, r], RS),
                 pl.multiple_of(rc_ref[t, r], RS)) for t in range(TOPK)]
        kgs.append(jnp.concatenate(
            [region_tile(k_ref, ro, co) for ro, co in offs], axis=0))
        vgs.append(jnp.concatenate(
            [region_tile(v_ref, ro, co) for ro, co in offs], axis=0))
    KG = jnp.stack(kgs, axis=0)
    VG = jnp.stack(vgs, axis=0)
    Q = (q_ref[0].reshape(BANDS, RS, N_WIN, RS, HEAD_DIM)
         .transpose(0, 2, 1, 3, 4).reshape(nreg, RSS, HEAD_DIM))
    S = jax.lax.dot_general(Q, KG, (((2,), (2,)), ((0,), (0,))),
                            preferred_element_type=jnp.float32)
    m = jnp.max(S, axis=2, keepdims=True)
    e = jnp.exp(S - m)
    ssum = jnp.sum(e, axis=2, keepdims=True)
    O = jax.lax.dot_general(e.astype(jnp.bfloat16), VG,
                            (((2,), (1,)), ((0,), (0,))),
                            preferred_element_type=jnp.float32) / ssum
    o_ref[0] = (O.astype(jnp.bfloat16)
                .reshape(BANDS, N_WIN, RS, RS, HEAD_DIM)
                .transpose(0, 2, 1, 3, 4).reshape(BANDS * RS, W, HEAD_DIM))


def _attention(ri, rc, q4, k4, v4):
    grid_spec = pltpu.PrefetchScalarGridSpec(
        num_scalar_prefetch=2,
        grid=(NUM_HEADS, N_WIN // BANDS),
        in_specs=[
            pl.BlockSpec((1, BANDS * RS, W, HEAD_DIM),
                         lambda h, i, *_: (h, i, 0, 0)),
            pl.BlockSpec((1, H, W, HEAD_DIM), lambda h, i, *_: (h, 0, 0, 0)),
            pl.BlockSpec((1, H, W, HEAD_DIM), lambda h, i, *_: (h, 0, 0, 0)),
        ],
        out_specs=pl.BlockSpec((1, BANDS * RS, W, HEAD_DIM),
                               lambda h, i, *_: (h, i, 0, 0)),
    )
    return pl.pallas_call(
        _attn_body,
        grid_spec=grid_spec,
        out_shape=jax.ShapeDtypeStruct((NUM_HEADS, H, W, HEAD_DIM), jnp.bfloat16),
    )(ri, rc, q4, k4, v4)


RCH = 8


def _out_body(a_ref, vp_ref, wh_ref, wl_ref, w9_ref, lb_ref, b_ref, o_ref,
              buf, sem):
    i = pl.program_id(0)

    def dma(slot, blk):
        return pltpu.make_async_copy(vp_ref.at[pl.ds(blk * RCH, RCH + 2)],
                                     buf.at[slot], sem.at[slot])

    @pl.when(i == 0)
    def _():
        dma(0, 0).start()

    @pl.when(i + 1 < H // RCH)
    def _():
        dma((i + 1) % 2, i + 1).start()

    dma(i % 2, i).wait()
    rows10 = buf[i % 2].astype(jnp.float32)
    lep = jnp.zeros((RCH, W, DIM), jnp.float32) + lb_ref[0]
    for dy in range(3):
        for dx in range(3):
            lep = lep + rows10[dy:dy + RCH, dx:dx + W, :] * w9_ref[dy * 3 + dx]
    l16 = lep.astype(jnp.bfloat16)
    for h in range(RCH):
        acc = jax.lax.dot_general(wl_ref[:], l16[h], (((0,), (1,)), ((), ())),
                                  preferred_element_type=jnp.float32)
        for hd in range(NUM_HEADS):
            acc = acc + jax.lax.dot_general(
                wh_ref[hd], a_ref[hd, h], (((0,), (1,)), ((), ())),
                preferred_element_type=jnp.float32)
        o_ref[h] = acc + b_ref[:]


def _out_proj(attn4, vp, wh, wl, w9, lb, b):
    return pl.pallas_call(
        _out_body,
        grid=(H // RCH,),
        in_specs=[
            pl.BlockSpec((NUM_HEADS, RCH, W, HEAD_DIM), lambda i: (0, i, 0, 0)),
            pl.BlockSpec(memory_space=pl.ANY),
            pl.BlockSpec((NUM_HEADS, HEAD_DIM, DIM), lambda i: (0, 0, 0)),
            pl.BlockSpec((DIM, DIM), lambda i: (0, 0)),
            pl.BlockSpec((9, DIM), lambda i: (0, 0)),
            pl.BlockSpec((1, DIM), lambda i: (0, 0)),
            pl.BlockSpec((DIM, 1), lambda i: (0, 0)),
        ],
        out_specs=pl.BlockSpec((RCH, DIM, W), lambda i: (i, 0, 0)),
        out_shape=jax.ShapeDtypeStruct((H, DIM, W), jnp.float32),
        scratch_shapes=[
            pltpu.VMEM((2, RCH + 2, W + 2, DIM), jnp.bfloat16),
            pltpu.SemaphoreType.DMA((2,)),
        ],
    )(attn4, vp, wh, wl, w9, lb, b)


def kernel(x, qkv_w, qkv_b, lepe_w, lepe_b, out_w, out_b):
    x_t = jnp.transpose(x, (0, 1, 3, 2)).reshape(H, DIM, W)

    q4, k4, v4, v_sp, qr, kr = _qkv_proj(x_t, qkv_w.T, qkv_b.reshape(1, -1))
    ri, rc = _route(qr, kr)

    q4 = q4.reshape(NUM_HEADS, H, W, HEAD_DIM)
    k4 = k4.reshape(NUM_HEADS, H, W, HEAD_DIM)
    v4 = v4.reshape(NUM_HEADS, H, W, HEAD_DIM)
    attn4 = _attention(ri, rc, q4, k4, v4)

    vp = jnp.pad(v_sp.reshape(H, W, DIM), ((1, 15), (1, 1), (0, 0)))

    wT16 = out_w.T.astype(jnp.bfloat16)
    wh = wT16.reshape(NUM_HEADS, HEAD_DIM, DIM)
    out = _out_proj(attn4, vp, wh, wT16, lepe_w.reshape(DIM, 9).T,
                    lepe_b.reshape(1, -1), out_b.reshape(-1, 1))
    return jnp.transpose(out.reshape(1, H, DIM, W), (0, 1, 3, 2))
